# baseline (speedup 1.0000x reference)
import jax
import jax.numpy as jnp
from jax import lax
from jax.experimental import pallas as pl
from jax.experimental.pallas import tpu as pltpu
from jax.experimental.pallas import tpu_sc as plsc

N_NODES = 100000
NP = 100096
CH = NP // 16
N_EDGES = 1600000
EPS = N_EDGES // 16
BC = 2048
NCM = 48

MUL = -1640531535
XORC = -1640531527
LMUL = 1000003


def _wl_body(edge_pos, edge_neg, out_pos, out_neg,
             labels_v, eb0, eb1, ssrc_v0, ssrc_v1,
             h_v0, h_v1, agg_v, agg_s,
             sem_in0, sem_in1, sem_sc0, sem_sc1):
    c = lax.axis_index("c")
    s = lax.axis_index("s")
    node_lo = s * CH
    ebb = (eb0, eb1)
    ssrcb = (ssrc_v0, ssrc_v1)
    hb = (h_v0, h_v1)
    semin = (sem_in0, sem_in1)
    semsc = (sem_sc0, sem_sc1)
    lanes = lax.iota(jnp.int32, 16)

    b_lo = (12500 * s) // 16 * 128
    b_hi = (12500 * (s + 1)) // 16 * 128
    thr = NCM * BC + BC - (b_hi - b_lo)

    def fill(ref, n16, value):
        def f(i, _):
            ref[pl.ds(i * 16, 16)] = jnp.full((16,), value, jnp.int32)
            return 0
        lax.fori_loop(0, n16, f, 0)

    def drain_sc(out_ref, b):
        pltpu.make_async_copy(out_ref.at[pl.ds(0, BC)], hb[b],
                              semsc[b]).wait()

    def run(edges, out_ref):
        fill(agg_v, CH // 16, 0)
        pltpu.sync_copy(agg_v, agg_s.at[pl.ds(node_lo, CH)])
        plsc.subcore_barrier()

        fill(h_v0, BC // 16, XORC)
        fill(h_v1, BC // 16, XORC)
        for b in range(2):
            pltpu.async_copy(edges.at[:, pl.ds(b_lo + b * BC, BC)],
                             ebb[b], semin[b])

        def hist_chunk(k2, _):
            for b in range(2):
                k = k2 * 2 + b
                pltpu.make_async_copy(edges.at[:, pl.ds(b_lo + k * BC, BC)],
                                      ebb[b], semin[b]).wait()

                @pl.when(k2 > 0)
                def _():
                    drain_sc(out_ref, b)

                def cp(i, _, b=b):
                    q = i * 64
                    for u in range(4):
                        o = q + u * 16
                        ssrcb[b][pl.ds(o, 16)] = ebb[b][0, pl.ds(o, 16)]
                    return 0
                lax.fori_loop(0, BC // 64, cp, 0)

                @pl.when(k + 2 < NCM)
                def _():
                    pltpu.async_copy(
                        edges.at[:, pl.ds(b_lo + (k + 2) * BC, BC)],
                        ebb[b], semin[b])
                pltpu.async_copy(hb[b], agg_s.at[ssrcb[b]], semsc[b], add=True)
            return 0
        lax.fori_loop(0, NCM // 2, hist_chunk, 0)
        for b in range(2):
            drain_sc(out_ref, b)

        pltpu.sync_copy(edges.at[:, pl.ds(b_hi - BC, BC)], eb0)

        def tail1(i, _):
            q = i * 16
            g = q + lanes
            ssrc_v0[pl.ds(q, 16)] = eb0[0, pl.ds(q, 16)]
            h_v0[pl.ds(q, 16)] = jnp.where(g >= thr, XORC, 0)
            return 0
        lax.fori_loop(0, BC // 16, tail1, 0)
        pltpu.sync_copy(h_v0, agg_s.at[ssrc_v0], add=True)
        plsc.subcore_barrier()

        pltpu.sync_copy(agg_s.at[pl.ds(node_lo, CH)], agg_v)
        pltpu.sync_copy(agg_v, out_ref.at[pl.ds(node_lo, CH)])
        fill(agg_v, CH // 16, 0)
        pltpu.sync_copy(agg_v, agg_s.at[pl.ds(node_lo, CH)])
        plsc.subcore_barrier()

        for it in range(2):
            pltpu.sync_copy(out_ref, labels_v)
            for b in range(2):
                pltpu.async_copy(edges.at[:, pl.ds(b_lo + b * BC, BC)],
                                 ebb[b], semin[b])

            def agg_chunk(k2, _):
                for b in range(2):
                    k = k2 * 2 + b
                    pltpu.make_async_copy(
                        edges.at[:, pl.ds(b_lo + k * BC, BC)],
                        ebb[b], semin[b]).wait()

                    @pl.when(k2 > 0)
                    def _():
                        drain_sc(out_ref, b)

                    def g(i, _, b=b):
                        q = i * 64
                        for u in range(4):
                            o = q + u * 16
                            d = ebb[b][1, pl.ds(o, 16)]
                            neigh = plsc.load_gather(labels_v, [d])
                            hb[b][pl.ds(o, 16)] = (neigh * MUL) ^ XORC
                            ssrcb[b][pl.ds(o, 16)] = ebb[b][0, pl.ds(o, 16)]
                        return 0
                    lax.fori_loop(0, BC // 64, g, 0)

                    @pl.when(k + 2 < NCM)
                    def _():
                        pltpu.async_copy(
                            edges.at[:, pl.ds(b_lo + (k + 2) * BC, BC)],
                            ebb[b], semin[b])
                    pltpu.async_copy(hb[b], agg_s.at[ssrcb[b]], semsc[b],
                                     add=True)
                return 0
            lax.fori_loop(0, NCM // 2, agg_chunk, 0)
            for b in range(2):
                drain_sc(out_ref, b)

            pltpu.sync_copy(edges.at[:, pl.ds(b_hi - BC, BC)], eb0)

            def tail23(i, _):
                q = i * 16
                g = q + lanes
                d = eb0[1, pl.ds(q, 16)]
                neigh = plsc.load_gather(labels_v, [d])
                h = (neigh * MUL) ^ XORC
                h_v0[pl.ds(q, 16)] = jnp.where(g >= thr, h, 0)
                ssrc_v0[pl.ds(q, 16)] = eb0[0, pl.ds(q, 16)]
                return 0
            lax.fori_loop(0, BC // 16, tail23, 0)
            pltpu.sync_copy(h_v0, agg_s.at[ssrc_v0], add=True)
            plsc.subcore_barrier()

            pltpu.sync_copy(agg_s.at[pl.ds(node_lo, CH)], agg_v)

            def upd(j, _):
                lo = labels_v[pl.ds(node_lo + j * 16, 16)]
                a = agg_v[pl.ds(j * 16, 16)]
                agg_v[pl.ds(j * 16, 16)] = lo * LMUL + a
                return 0
            lax.fori_loop(0, CH // 16, upd, 0)
            pltpu.sync_copy(agg_v, out_ref.at[pl.ds(node_lo, CH)])
            if it == 0:
                fill(agg_v, CH // 16, 0)
                pltpu.sync_copy(agg_v, agg_s.at[pl.ds(node_lo, CH)])
            plsc.subcore_barrier()

    @pl.when(c == 0)
    def _():
        run(edge_pos, out_pos)

    @pl.when(c == 1)
    def _():
        run(edge_neg, out_neg)


R = 2048
RB = R // 16
NV = CH // 16


def _relabel_body(lab_pos, lab_neg, out_pos, out_neg,
                  key_v, val_v, ext_v, pos_v, rank_v, hist_v, offs_v,
                  blk_v, stage_v, start_v, tbuf_v,
                  keys_a, vals_a, keys_b, vals_b, hists_s, offs_s, tsum_s,
                  inv_s):
    c = lax.axis_index("c")
    s = lax.axis_index("s")
    node_lo = s * CH
    lanes = lax.iota(jnp.int32, 16)

    def digits(k, sh, m):
        u = plsc.bitcast(k, jnp.uint32) >> sh
        return plsc.bitcast(u, jnp.int32) & m

    def one_pass(lab, srck, srcv, dstk, dstv, sh, m, first):
        if first:
            pltpu.sync_copy(lab.at[pl.ds(node_lo, CH)], key_v)

            def fix(i, _):
                g = node_lo + i * 16 + lanes
                k = key_v[pl.ds(i * 16, 16)]
                key_v[pl.ds(i * 16, 16)] = jnp.where(g < N_NODES, k, -1)
                val_v[pl.ds(i * 16, 16)] = g
                return 0
            lax.fori_loop(0, NV, fix, 0)
        else:
            pltpu.sync_copy(srck.at[pl.ds(node_lo, CH)], key_v)
            pltpu.sync_copy(srcv.at[pl.ds(node_lo, CH)], val_v)

        def z(i, _):
            hist_v[pl.ds(i * 16, 16)] = jnp.zeros((16,), jnp.int32)
            return 0
        lax.fori_loop(0, R // 16, z, 0)

        def hist(i, _):
            d = digits(key_v[pl.ds(i * 16, 16)], sh, m)
            cnt, last = plsc.scan_count(d)
            plsc.addupdate_scatter(hist_v, [d], cnt, mask=last)
            return 0
        lax.fori_loop(0, NV, hist, 0)
        pltpu.sync_copy(hist_v, hists_s.at[pl.ds(s * R, R)])
        plsc.subcore_barrier()

        for t2 in range(16):
            pltpu.sync_copy(hists_s.at[pl.ds(t2 * R + s * RB, RB)],
                            blk_v.at[pl.ds(t2 * RB, RB)])
        carry = jnp.int32(0)
        for j in range(RB // 16):
            run = jnp.zeros((16,), jnp.int32)
            for t2 in range(16):
                h = blk_v[pl.ds(t2 * RB + j * 16, 16)]
                stage_v[pl.ds(t2 * RB + j * 16, 16)] = run
                run = run + h
            excl = plsc.cumsum(run) - run + carry
            start_v[pl.ds(j * 16, 16)] = excl
            carry = carry + lax.reduce_sum(run, (0,))
        tbuf_v[pl.ds(0, 16)] = jnp.full((16,), carry, jnp.int32)
        pltpu.sync_copy(tbuf_v, tsum_s.at[pl.ds(s * 16, 16)])
        plsc.subcore_barrier()
        pltpu.sync_copy(tsum_s, blk_v.at[pl.ds(0, 256)])
        totals = plsc.load_gather(blk_v, [lanes * 16])
        base = lax.reduce_sum(jnp.where(lanes < s, totals, 0), (0,))

        def addb(j, _):
            sv = start_v[pl.ds(j * 16, 16)] + base

            def addt(t2, _):
                q = t2 * RB + j * 16
                stage_v[pl.ds(q, 16)] = stage_v[pl.ds(q, 16)] + sv
                return 0
            lax.fori_loop(0, 16, addt, 0)
            return 0
        lax.fori_loop(0, RB // 16, addb, 0)
        for t2 in range(16):
            pltpu.sync_copy(stage_v.at[pl.ds(t2 * RB, RB)],
                            offs_s.at[pl.ds(t2 * R + s * RB, RB)])
        plsc.subcore_barrier()
        pltpu.sync_copy(offs_s.at[pl.ds(s * R, R)], offs_v)

        def place(i, _):
            d = digits(key_v[pl.ds(i * 16, 16)], sh, m)
            cnt, last = plsc.scan_count(d)
            bb = plsc.load_gather(offs_v, [d])
            pos_v[pl.ds(i * 16, 16)] = bb + cnt - 1
            plsc.addupdate_scatter(offs_v, [d], cnt, mask=last)
            return 0
        lax.fori_loop(0, NV, place, 0)
        pltpu.sync_copy(key_v, dstk.at[pos_v])
        pltpu.sync_copy(val_v, dstv.at[pos_v])
        plsc.subcore_barrier()

    def run(lab, out_ref):
        one_pass(lab, None, None, keys_a, vals_a, 0, R - 1, True)
        one_pass(lab, keys_a, vals_a, keys_b, vals_b, 11, R - 1, False)
        one_pass(lab, keys_b, vals_b, keys_a, vals_a, 22, 1023, False)

        pltpu.sync_copy(keys_a.at[pl.ds(node_lo, CH)], ext_v.at[pl.ds(16, CH)])

        @pl.when(s > 0)
        def _():
            pltpu.sync_copy(keys_a.at[pl.ds(node_lo - 16, 16)],
                            ext_v.at[pl.ds(0, 16)])

        def flags(i, carry):
            k = ext_v[pl.ds(16 + i * 16, 16)]
            prev = ext_v[pl.ds(15 + i * 16, 16)]
            g = node_lo + i * 16 + lanes
            f = jnp.where((g != 0) & (k != prev), 1, 0)
            pos_v[pl.ds(i * 16, 16)] = plsc.cumsum(f) + carry
            return carry + lax.reduce_sum(f, (0,))
        t = lax.fori_loop(0, NV, flags, jnp.int32(0))
        tbuf_v[pl.ds(0, 16)] = jnp.full((16,), t, jnp.int32)
        pltpu.sync_copy(tbuf_v, tsum_s.at[pl.ds(s * 16, 16)])
        plsc.subcore_barrier()
        pltpu.sync_copy(tsum_s, blk_v.at[pl.ds(0, 256)])
        totals = plsc.load_gather(blk_v, [lanes * 16])
        base = lax.reduce_sum(jnp.where(lanes < s, totals, 0), (0,))
        pltpu.sync_copy(vals_a.at[pl.ds(node_lo, CH)], val_v)

        def mkrank(i, _):
            r = pos_v[pl.ds(i * 16, 16)] + base
            rank_v[pl.ds(i * 16, 16)] = r.astype(jnp.float32)
            return 0
        lax.fori_loop(0, NV, mkrank, 0)
        pltpu.sync_copy(rank_v, inv_s.at[val_v])
        plsc.subcore_barrier()
        pltpu.sync_copy(inv_s.at[pl.ds(node_lo, CH)], rank_v)
        pltpu.sync_copy(rank_v, out_ref.at[pl.ds(node_lo, CH)])

    @pl.when(c == 0)
    def _():
        run(lab_pos, out_pos)

    @pl.when(c == 1)
    def _():
        run(lab_neg, out_neg)


def kernel(feature_a, feature_b, edge_index_pos, edge_index_neg):
    mesh = plsc.VectorSubcoreMesh(core_axis_name="c", subcore_axis_name="s")
    wl = pl.kernel(
        _wl_body,
        out_type=(jax.ShapeDtypeStruct((NP,), jnp.int32),
                  jax.ShapeDtypeStruct((NP,), jnp.int32)),
        mesh=mesh,
        compiler_params=pltpu.CompilerParams(needs_layout_passes=False),
        scratch_types=[
            pltpu.VMEM((NP,), jnp.int32),
            pltpu.VMEM((2, BC), jnp.int32),
            pltpu.VMEM((2, BC), jnp.int32),
            pltpu.VMEM((BC,), jnp.int32),
            pltpu.VMEM((BC,), jnp.int32),
            pltpu.VMEM((BC,), jnp.int32),
            pltpu.VMEM((BC,), jnp.int32),
            pltpu.VMEM((CH,), jnp.int32),
            pltpu.VMEM_SHARED((NP,), jnp.int32),
            pltpu.SemaphoreType.DMA,
            pltpu.SemaphoreType.DMA,
            pltpu.SemaphoreType.DMA,
            pltpu.SemaphoreType.DMA,
        ],
    )
    relab = pl.kernel(
        _relabel_body,
        out_type=(jax.ShapeDtypeStruct((NP,), jnp.float32),
                  jax.ShapeDtypeStruct((NP,), jnp.float32)),
        mesh=mesh,
        compiler_params=pltpu.CompilerParams(needs_layout_passes=False),
        scratch_types=[
            pltpu.VMEM((CH,), jnp.int32),
            pltpu.VMEM((CH,), jnp.int32),
            pltpu.VMEM((CH + 16,), jnp.int32),
            pltpu.VMEM((CH,), jnp.int32),
            pltpu.VMEM((CH,), jnp.float32),
            pltpu.VMEM((R,), jnp.int32),
            pltpu.VMEM((R,), jnp.int32),
            pltpu.VMEM((R,), jnp.int32),
            pltpu.VMEM((R,), jnp.int32),
            pltpu.VMEM((RB,), jnp.int32),
            pltpu.VMEM((16,), jnp.int32),
            pltpu.VMEM_SHARED((NP,), jnp.int32),
            pltpu.VMEM_SHARED((NP,), jnp.int32),
            pltpu.VMEM_SHARED((NP,), jnp.int32),
            pltpu.VMEM_SHARED((NP,), jnp.int32),
            pltpu.VMEM_SHARED((16 * R,), jnp.int32),
            pltpu.VMEM_SHARED((16 * R,), jnp.int32),
            pltpu.VMEM_SHARED((256,), jnp.int32),
            pltpu.VMEM_SHARED((NP,), jnp.float32),
        ],
    )
    lab_pos, lab_neg = wl(edge_index_pos, edge_index_neg)
    inv_pos, inv_neg = relab(lab_pos, lab_neg)
    tp = inv_pos[:N_NODES].reshape(-1, 1)
    tn = inv_neg[:N_NODES].reshape(-1, 1)
    feats = jnp.concatenate([feature_a, feature_b], axis=0)
    return jnp.concatenate([feats, tp, tn], axis=1)


# speedup vs baseline: 1.0758x; 1.0758x over previous
import jax
import jax.numpy as jnp
from jax import lax
from jax.experimental import pallas as pl
from jax.experimental.pallas import tpu as pltpu
from jax.experimental.pallas import tpu_sc as plsc

N_NODES = 100000
NP = 100096
CH = NP // 16
N_EDGES = 1600000
EPS = N_EDGES // 16
B = 2000
NCHUNK = EPS // B

MUL = -1640531535
XORC = -1640531527
LMUL = 1000003


def _wl_body(src_pos, dst_pos, src_neg, dst_neg, out_pos, out_neg,
             labels_v, src_v0, src_v1, dst_v0, dst_v1, ssrc_v0, ssrc_v1,
             h_v0, h_v1, agg_v, agg_s,
             sem_in0, sem_in1, sem_sc0, sem_sc1):
    c = lax.axis_index("c")
    s = lax.axis_index("s")
    node_lo = s * CH
    base_e = s * EPS
    srcb = (src_v0, src_v1)
    dstb = (dst_v0, dst_v1)
    ssrcb = (ssrc_v0, ssrc_v1)
    hb = (h_v0, h_v1)
    semin = (sem_in0, sem_in1)
    semsc = (sem_sc0, sem_sc1)

    def fill(ref, n16, value):
        def f(i, _):
            ref[pl.ds(i * 16, 16)] = jnp.full((16,), value, jnp.int32)
            return 0
        lax.fori_loop(0, n16, f, 0)

    def run(srcs, dsts, out_ref):
        fill(agg_v, CH // 16, 0)
        pltpu.sync_copy(agg_v, agg_s.at[pl.ds(node_lo, CH)])
        plsc.subcore_barrier()

        fill(h_v0, B // 16, XORC)
        fill(h_v1, B // 16, XORC)
        for b in range(2):
            pltpu.async_copy(srcs.at[pl.ds(base_e + b * B, B)],
                             srcb[b], semin[b])

        def hist_chunk(k2, _):
            for b in range(2):
                k = k2 * 2 + b
                pltpu.make_async_copy(srcs.at[pl.ds(base_e + k * B, B)],
                                      srcb[b], semin[b]).wait()

                @pl.when(k2 > 0)
                def _():
                    pltpu.make_async_copy(srcs.at[pl.ds(0, B)], hb[b],
                                          semsc[b]).wait()

                def cp(i, _, b=b):
                    q = i * 80
                    for u in range(5):
                        ssrcb[b][pl.ds(q + u * 16, 16)] = \
                            srcb[b][pl.ds(q + u * 16, 16)]
                    return 0
                lax.fori_loop(0, B // 80, cp, 0)

                @pl.when(k + 2 < NCHUNK)
                def _():
                    pltpu.async_copy(srcs.at[pl.ds(base_e + (k + 2) * B, B)],
                                     srcb[b], semin[b])
                pltpu.async_copy(hb[b], agg_s.at[ssrcb[b]], semsc[b], add=True)
            return 0
        lax.fori_loop(0, NCHUNK // 2, hist_chunk, 0)
        for b in range(2):
            pltpu.make_async_copy(srcs.at[pl.ds(0, B)], hb[b],
                                  semsc[b]).wait()
        plsc.subcore_barrier()

        pltpu.sync_copy(agg_s.at[pl.ds(node_lo, CH)], agg_v)
        pltpu.sync_copy(agg_v, out_ref.at[pl.ds(node_lo, CH)])
        fill(agg_v, CH // 16, 0)
        pltpu.sync_copy(agg_v, agg_s.at[pl.ds(node_lo, CH)])
        plsc.subcore_barrier()

        for it in range(2):
            pltpu.sync_copy(out_ref, labels_v)
            for b in range(2):
                pltpu.async_copy(srcs.at[pl.ds(base_e + b * B, B)],
                                 srcb[b], semin[b])
                pltpu.async_copy(dsts.at[pl.ds(base_e + b * B, B)],
                                 dstb[b], semin[b])

            def agg_chunk(k2, _):
                for b in range(2):
                    k = k2 * 2 + b
                    pltpu.make_async_copy(srcs.at[pl.ds(base_e + k * B, B)],
                                          srcb[b], semin[b]).wait()
                    pltpu.make_async_copy(dsts.at[pl.ds(base_e + k * B, B)],
                                          dstb[b], semin[b]).wait()

                    @pl.when(k2 > 0)
                    def _():
                        pltpu.make_async_copy(srcs.at[pl.ds(0, B)], hb[b],
                                              semsc[b]).wait()

                    def g(i, _, b=b):
                        q = i * 80
                        for u in range(5):
                            o = q + u * 16
                            d = dstb[b][pl.ds(o, 16)]
                            neigh = plsc.load_gather(labels_v, [d])
                            hb[b][pl.ds(o, 16)] = (neigh * MUL) ^ XORC
                            ssrcb[b][pl.ds(o, 16)] = srcb[b][pl.ds(o, 16)]
                        return 0
                    lax.fori_loop(0, B // 80, g, 0)

                    @pl.when(k + 2 < NCHUNK)
                    def _():
                        nb = base_e + (k + 2) * B
                        pltpu.async_copy(srcs.at[pl.ds(nb, B)],
                                         srcb[b], semin[b])
                        pltpu.async_copy(dsts.at[pl.ds(nb, B)],
                                         dstb[b], semin[b])
                    pltpu.async_copy(hb[b], agg_s.at[ssrcb[b]], semsc[b],
                                     add=True)
                return 0
            lax.fori_loop(0, NCHUNK // 2, agg_chunk, 0)
            for b in range(2):
                pltpu.make_async_copy(srcs.at[pl.ds(0, B)], hb[b],
                                      semsc[b]).wait()
            plsc.subcore_barrier()

            pltpu.sync_copy(agg_s.at[pl.ds(node_lo, CH)], agg_v)

            def upd(j, _):
                lo = labels_v[pl.ds(node_lo + j * 16, 16)]
                a = agg_v[pl.ds(j * 16, 16)]
                agg_v[pl.ds(j * 16, 16)] = lo * LMUL + a
                return 0
            lax.fori_loop(0, CH // 16, upd, 0)
            pltpu.sync_copy(agg_v, out_ref.at[pl.ds(node_lo, CH)])
            if it == 0:
                fill(agg_v, CH // 16, 0)
                pltpu.sync_copy(agg_v, agg_s.at[pl.ds(node_lo, CH)])
            plsc.subcore_barrier()

    @pl.when(c == 0)
    def _():
        run(src_pos, dst_pos, out_pos)

    @pl.when(c == 1)
    def _():
        run(src_neg, dst_neg, out_neg)


R = 2048
RB = R // 16
NV = CH // 16


def _relabel_body(lab_pos, lab_neg, out_pos, out_neg,
                  key_v, val_v, ext_v, pos_v, rank_v, hist_v, offs_v,
                  blk_v, stage_v, start_v, tbuf_v,
                  keys_a, vals_a, keys_b, vals_b, hists_s, offs_s, tsum_s,
                  inv_s):
    c = lax.axis_index("c")
    s = lax.axis_index("s")
    node_lo = s * CH
    lanes = lax.iota(jnp.int32, 16)

    def digits(k, sh, m):
        u = plsc.bitcast(k, jnp.uint32) >> sh
        return plsc.bitcast(u, jnp.int32) & m

    def one_pass(lab, srck, srcv, dstk, dstv, sh, m, first):
        if first:
            pltpu.sync_copy(lab.at[pl.ds(node_lo, CH)], key_v)

            def fix(i, _):
                g = node_lo + i * 16 + lanes
                k = key_v[pl.ds(i * 16, 16)]
                key_v[pl.ds(i * 16, 16)] = jnp.where(g < N_NODES, k, -1)
                val_v[pl.ds(i * 16, 16)] = g
                return 0
            lax.fori_loop(0, NV, fix, 0)
        else:
            pltpu.sync_copy(srck.at[pl.ds(node_lo, CH)], key_v)
            pltpu.sync_copy(srcv.at[pl.ds(node_lo, CH)], val_v)

        def z(i, _):
            hist_v[pl.ds(i * 16, 16)] = jnp.zeros((16,), jnp.int32)
            return 0
        lax.fori_loop(0, R // 16, z, 0)

        def hist(i, _):
            d = digits(key_v[pl.ds(i * 16, 16)], sh, m)
            cnt, last = plsc.scan_count(d)
            plsc.addupdate_scatter(hist_v, [d], cnt, mask=last)
            return 0
        lax.fori_loop(0, NV, hist, 0)
        pltpu.sync_copy(hist_v, hists_s.at[pl.ds(s * R, R)])
        plsc.subcore_barrier()

        for t2 in range(16):
            pltpu.sync_copy(hists_s.at[pl.ds(t2 * R + s * RB, RB)],
                            blk_v.at[pl.ds(t2 * RB, RB)])
        carry = jnp.int32(0)
        for j in range(RB // 16):
            run = jnp.zeros((16,), jnp.int32)
            for t2 in range(16):
                h = blk_v[pl.ds(t2 * RB + j * 16, 16)]
                stage_v[pl.ds(t2 * RB + j * 16, 16)] = run
                run = run + h
            excl = plsc.cumsum(run) - run + carry
            start_v[pl.ds(j * 16, 16)] = excl
            carry = carry + lax.reduce_sum(run, (0,))
        tbuf_v[pl.ds(0, 16)] = jnp.full((16,), carry, jnp.int32)
        pltpu.sync_copy(tbuf_v, tsum_s.at[pl.ds(s * 16, 16)])
        plsc.subcore_barrier()
        pltpu.sync_copy(tsum_s, blk_v.at[pl.ds(0, 256)])
        totals = plsc.load_gather(blk_v, [lanes * 16])
        base = lax.reduce_sum(jnp.where(lanes < s, totals, 0), (0,))

        def addb(j, _):
            sv = start_v[pl.ds(j * 16, 16)] + base

            def addt(t2, _):
                q = t2 * RB + j * 16
                stage_v[pl.ds(q, 16)] = stage_v[pl.ds(q, 16)] + sv
                return 0
            lax.fori_loop(0, 16, addt, 0)
            return 0
        lax.fori_loop(0, RB // 16, addb, 0)
        for t2 in range(16):
            pltpu.sync_copy(stage_v.at[pl.ds(t2 * RB, RB)],
                            offs_s.at[pl.ds(t2 * R + s * RB, RB)])
        plsc.subcore_barrier()
        pltpu.sync_copy(offs_s.at[pl.ds(s * R, R)], offs_v)

        def place(i, _):
            d = digits(key_v[pl.ds(i * 16, 16)], sh, m)
            cnt, last = plsc.scan_count(d)
            bb = plsc.load_gather(offs_v, [d])
            pos_v[pl.ds(i * 16, 16)] = bb + cnt - 1
            plsc.addupdate_scatter(offs_v, [d], cnt, mask=last)
            return 0
        lax.fori_loop(0, NV, place, 0)
        pltpu.sync_copy(key_v, dstk.at[pos_v])
        pltpu.sync_copy(val_v, dstv.at[pos_v])
        plsc.subcore_barrier()

    def run(lab, out_ref):
        one_pass(lab, None, None, keys_a, vals_a, 0, R - 1, True)
        one_pass(lab, keys_a, vals_a, keys_b, vals_b, 11, R - 1, False)
        one_pass(lab, keys_b, vals_b, keys_a, vals_a, 22, 1023, False)

        pltpu.sync_copy(keys_a.at[pl.ds(node_lo, CH)], ext_v.at[pl.ds(16, CH)])

        @pl.when(s > 0)
        def _():
            pltpu.sync_copy(keys_a.at[pl.ds(node_lo - 16, 16)],
                            ext_v.at[pl.ds(0, 16)])

        def flags(i, carry):
            k = ext_v[pl.ds(16 + i * 16, 16)]
            prev = ext_v[pl.ds(15 + i * 16, 16)]
            g = node_lo + i * 16 + lanes
            f = jnp.where((g != 0) & (k != prev), 1, 0)
            pos_v[pl.ds(i * 16, 16)] = plsc.cumsum(f) + carry
            return carry + lax.reduce_sum(f, (0,))
        t = lax.fori_loop(0, NV, flags, jnp.int32(0))
        tbuf_v[pl.ds(0, 16)] = jnp.full((16,), t, jnp.int32)
        pltpu.sync_copy(tbuf_v, tsum_s.at[pl.ds(s * 16, 16)])
        plsc.subcore_barrier()
        pltpu.sync_copy(tsum_s, blk_v.at[pl.ds(0, 256)])
        totals = plsc.load_gather(blk_v, [lanes * 16])
        base = lax.reduce_sum(jnp.where(lanes < s, totals, 0), (0,))
        pltpu.sync_copy(vals_a.at[pl.ds(node_lo, CH)], val_v)

        def mkrank(i, _):
            r = pos_v[pl.ds(i * 16, 16)] + base
            rank_v[pl.ds(i * 16, 16)] = r.astype(jnp.float32)
            return 0
        lax.fori_loop(0, NV, mkrank, 0)
        pltpu.sync_copy(rank_v, inv_s.at[val_v])
        plsc.subcore_barrier()
        pltpu.sync_copy(inv_s.at[pl.ds(node_lo, CH)], rank_v)
        pltpu.sync_copy(rank_v, out_ref.at[pl.ds(node_lo, CH)])

    @pl.when(c == 0)
    def _():
        run(lab_pos, out_pos)

    @pl.when(c == 1)
    def _():
        run(lab_neg, out_neg)


def kernel(feature_a, feature_b, edge_index_pos, edge_index_neg):
    mesh = plsc.VectorSubcoreMesh(core_axis_name="c", subcore_axis_name="s")
    wl = pl.kernel(
        _wl_body,
        out_type=(jax.ShapeDtypeStruct((NP,), jnp.int32),
                  jax.ShapeDtypeStruct((NP,), jnp.int32)),
        mesh=mesh,
        compiler_params=pltpu.CompilerParams(needs_layout_passes=False),
        scratch_types=[
            pltpu.VMEM((NP,), jnp.int32),
            pltpu.VMEM((B,), jnp.int32),
            pltpu.VMEM((B,), jnp.int32),
            pltpu.VMEM((B,), jnp.int32),
            pltpu.VMEM((B,), jnp.int32),
            pltpu.VMEM((B,), jnp.int32),
            pltpu.VMEM((B,), jnp.int32),
            pltpu.VMEM((B,), jnp.int32),
            pltpu.VMEM((B,), jnp.int32),
            pltpu.VMEM((CH,), jnp.int32),
            pltpu.VMEM_SHARED((NP,), jnp.int32),
            pltpu.SemaphoreType.DMA,
            pltpu.SemaphoreType.DMA,
            pltpu.SemaphoreType.DMA,
            pltpu.SemaphoreType.DMA,
        ],
    )
    relab = pl.kernel(
        _relabel_body,
        out_type=(jax.ShapeDtypeStruct((NP,), jnp.float32),
                  jax.ShapeDtypeStruct((NP,), jnp.float32)),
        mesh=mesh,
        compiler_params=pltpu.CompilerParams(needs_layout_passes=False),
        scratch_types=[
            pltpu.VMEM((CH,), jnp.int32),
            pltpu.VMEM((CH,), jnp.int32),
            pltpu.VMEM((CH + 16,), jnp.int32),
            pltpu.VMEM((CH,), jnp.int32),
            pltpu.VMEM((CH,), jnp.float32),
            pltpu.VMEM((R,), jnp.int32),
            pltpu.VMEM((R,), jnp.int32),
            pltpu.VMEM((R,), jnp.int32),
            pltpu.VMEM((R,), jnp.int32),
            pltpu.VMEM((RB,), jnp.int32),
            pltpu.VMEM((16,), jnp.int32),
            pltpu.VMEM_SHARED((NP,), jnp.int32),
            pltpu.VMEM_SHARED((NP,), jnp.int32),
            pltpu.VMEM_SHARED((NP,), jnp.int32),
            pltpu.VMEM_SHARED((NP,), jnp.int32),
            pltpu.VMEM_SHARED((16 * R,), jnp.int32),
            pltpu.VMEM_SHARED((16 * R,), jnp.int32),
            pltpu.VMEM_SHARED((256,), jnp.int32),
            pltpu.VMEM_SHARED((NP,), jnp.float32),
        ],
    )
    ep = edge_index_pos.astype(jnp.int32)
    en = edge_index_neg.astype(jnp.int32)
    lab_pos, lab_neg = wl(ep[0], ep[1], en[0], en[1])
    inv_pos, inv_neg = relab(lab_pos, lab_neg)
    tp = inv_pos[:N_NODES].reshape(-1, 1)
    tn = inv_neg[:N_NODES].reshape(-1, 1)
    feats = jnp.concatenate([feature_a, feature_b], axis=0)
    return jnp.concatenate([feats, tp, tn], axis=1)
